# Initial kernel scaffold; baseline (speedup 1.0000x reference)
import jax
import jax.numpy as jnp
from jax.experimental import pallas as pl


def kernel(contexts, context_utterance_lengths, context_lengths, queries, query_lengths, emb, Wx_u, Wh_u, b_u, Wx_c, Wh_c, b_c):
    raise NotImplementedError("write your pallas kernel here")


# speedup vs baseline: 3.3016x; 3.3016x over previous
import functools

import jax
import jax.numpy as jnp
from jax import lax
from jax.experimental import pallas as pl
from jax.experimental.pallas import tpu as pltpu
from jax.experimental.pallas import tpu_sc as plsc

V = 30000
D = 256
H = 256
B = 16
U = 15
W = 30
NROW = B * (U + 1)
P = NROW * W
NW = 32
RPW = P // NW
NPW = NROW // NW
CHUNK = 80
NCTX = B * U * W


def _sc_gather(toks, ctx_len, emb):
    mesh = plsc.VectorSubcoreMesh(core_axis_name="c", subcore_axis_name="s")

    @functools.partial(
        pl.kernel,
        out_type=jax.ShapeDtypeStruct((P, D), jnp.float32),
        mesh=mesh,
        compiler_params=pltpu.CompilerParams(needs_layout_passes=False),
        scratch_types=[
            pltpu.VMEM((P,), jnp.int32),
            pltpu.VMEM((128,), jnp.int32),
            pltpu.VMEM((RPW,), jnp.int32),
            pltpu.VMEM((RPW, D), jnp.float32),
            pltpu.SemaphoreType.DMA,
        ],
    )
    def sc_kernel(toks_hbm, len_hbm, emb_hbm, out_hbm, toks_v, len_v, idx_v,
                  rows_v, sem):
        wid = lax.axis_index("s") * 2 + lax.axis_index("c")
        n0 = wid * NPW
        pltpu.sync_copy(toks_hbm, toks_v)
        pltpu.sync_copy(len_hbm, len_v.at[pl.ds(0, B)])
        lane = lax.iota(jnp.int32, 16)
        us = n0 >> 4
        b = (n0 & 15) + (lane & 7)
        lb = plsc.load_gather(len_v, [b])
        src_u = jnp.where(us < lb, us, us - 1)
        off0 = jnp.where(
            lb == us,
            NCTX + b * W,
            b * (U * W) + src_u * W,
        )
        for i in range(RPW // 16):
            w = 2 * i + (lane >> 3)
            idx_v[pl.ds(i * 16, 16)] = plsc.load_gather(toks_v, [off0 + w])
        descs = [
            pltpu.async_copy(
                emb_hbm.at[idx_v.at[pl.ds(c * CHUNK, CHUNK)]],
                rows_v.at[pl.ds(c * CHUNK, CHUNK)],
                sem,
            )
            for c in range(RPW // CHUNK)
        ]
        for d in descs:
            d.wait()
        for w in range(W):
            pltpu.sync_copy(
                rows_v.at[pl.ds(w * NPW, NPW)],
                out_hbm.at[pl.ds(w * NROW + n0, NPW)],
            )

    return sc_kernel(toks, ctx_len, emb)


def _tc_gru(x_tm, lens, ctx_len, wx_u, wh_u, b_u, wx_c, wh_c, b_c):
    def tc_kernel(x_ref, len_ref, cl_ref, wxu_ref, whu_ref, bu_ref, wxc_ref,
                  whc_ref, bc_ref, out_ref, g2_ref):
        wxu = wxu_ref[...]
        whu = whu_ref[...]
        bu = bu_ref[...]
        lenv = len_ref[...]
        tsel = jnp.clip(lenv - 1, 0, W - 1)
        valid = lenv > 0

        def step(t, carry):
            h, uacc = carry
            xt = x_ref[t]
            gx = jnp.dot(xt, wxu, preferred_element_type=jnp.float32) + bu
            gh = jnp.dot(h, whu, preferred_element_type=jnp.float32)
            r = jax.nn.sigmoid(gx[:, :H] + gh[:, :H])
            z = jax.nn.sigmoid(gx[:, H:2 * H] + gh[:, H:2 * H])
            nn = jnp.tanh(gx[:, 2 * H:] + r * gh[:, 2 * H:])
            h = (1.0 - z) * nn + z * h
            sel = jnp.logical_and(tsel == t, valid)
            return h, jnp.where(sel, h, uacc)

        zeros = jnp.zeros((NROW, H), jnp.float32)
        _, uacc = lax.fori_loop(0, W, step, (zeros, zeros))

        g2_ref[...] = (jnp.dot(uacc, wxc_ref[...],
                               preferred_element_type=jnp.float32) + bc_ref[...])
        whc = whc_ref[...]
        clv = cl_ref[...]

        def step2(s, carry):
            h2, cacc = carry
            gx2 = g2_ref[pl.ds(s * B, B), :]
            gh2 = jnp.dot(h2, whc, preferred_element_type=jnp.float32)
            r2 = jax.nn.sigmoid(gx2[:, :H] + gh2[:, :H])
            z2 = jax.nn.sigmoid(gx2[:, H:2 * H] + gh2[:, H:2 * H])
            n2 = jnp.tanh(gx2[:, 2 * H:] + r2 * gh2[:, 2 * H:])
            h2 = (1.0 - z2) * n2 + z2 * h2
            return h2, jnp.where(clv == s, h2, cacc)

        z16 = jnp.zeros((B, H), jnp.float32)
        _, cacc = lax.fori_loop(0, U + 1, step2, (z16, z16))
        out_ref[...] = cacc

    return pl.pallas_call(
        tc_kernel,
        out_shape=jax.ShapeDtypeStruct((B, H), jnp.float32),
        scratch_shapes=[pltpu.VMEM((NROW, 3 * H), jnp.float32)],
    )(x_tm, lens, ctx_len, wx_u, wh_u, b_u, wx_c, wh_c, b_c)


def kernel(contexts, context_utterance_lengths, context_lengths, queries,
           query_lengths, emb, Wx_u, Wh_u, b_u, Wx_c, Wh_c, b_c):
    toks = jnp.concatenate([contexts.reshape(-1), queries.reshape(-1)])
    x = _sc_gather(toks, context_lengths, emb)
    x_tm = x.reshape(W, NROW, D)
    lens = jnp.concatenate(
        [context_utterance_lengths, query_lengths[:, None]], axis=1
    ).T.reshape(NROW, 1)
    return _tc_gru(x_tm, lens, context_lengths.reshape(B, 1),
                   Wx_u, Wh_u, b_u.reshape(1, 3 * H),
                   Wx_c, Wh_c, b_c.reshape(1, 3 * H))


# speedup vs baseline: 3.5927x; 1.0882x over previous
import functools

import jax
import jax.numpy as jnp
from jax import lax
from jax.experimental import pallas as pl
from jax.experimental.pallas import tpu as pltpu
from jax.experimental.pallas import tpu_sc as plsc

V = 30000
D = 256
H = 256
B = 16
U = 15
W = 30
NROW = B * (U + 1)
P = NROW * W
NW = 32
RPW = P // NW
NPW = NROW // NW
CHUNK = 80
NCTX = B * U * W


def _sc_gather(toks, ctx_len, emb):
    mesh = plsc.VectorSubcoreMesh(core_axis_name="c", subcore_axis_name="s")

    @functools.partial(
        pl.kernel,
        out_type=jax.ShapeDtypeStruct((P, D), jnp.float32),
        mesh=mesh,
        compiler_params=pltpu.CompilerParams(needs_layout_passes=False),
        scratch_types=[
            pltpu.VMEM((P,), jnp.int32),
            pltpu.VMEM((128,), jnp.int32),
            pltpu.VMEM((RPW,), jnp.int32),
            pltpu.VMEM((RPW, D), jnp.float32),
            pltpu.SemaphoreType.DMA,
        ],
    )
    def sc_kernel(toks_hbm, len_hbm, emb_hbm, out_hbm, toks_v, len_v, idx_v,
                  rows_v, sem):
        wid = lax.axis_index("s") * 2 + lax.axis_index("c")
        n0 = wid * NPW
        pltpu.sync_copy(toks_hbm, toks_v)
        pltpu.sync_copy(len_hbm, len_v.at[pl.ds(0, B)])
        lane = lax.iota(jnp.int32, 16)
        us = n0 >> 4
        b = (n0 & 15) + (lane & 7)
        lb = plsc.load_gather(len_v, [b])
        src_u = jnp.where(us < lb, us, us - 1)
        off0 = jnp.where(
            lb == us,
            NCTX + b * W,
            b * (U * W) + src_u * W,
        )
        for i in range(RPW // 16):
            w = 2 * i + (lane >> 3)
            idx_v[pl.ds(i * 16, 16)] = plsc.load_gather(toks_v, [off0 + w])
        descs = [
            pltpu.async_copy(
                emb_hbm.at[idx_v.at[pl.ds(c * CHUNK, CHUNK)]],
                rows_v.at[pl.ds(c * CHUNK, CHUNK)],
                sem,
            )
            for c in range(RPW // CHUNK)
        ]
        for d in descs:
            d.wait()
        for w in range(W):
            pltpu.sync_copy(
                rows_v.at[pl.ds(w * NPW, NPW)],
                out_hbm.at[pl.ds(w * NROW + n0, NPW)],
            )

    return sc_kernel(toks, ctx_len, emb)


def _tc_gru(x_tm, lens, ctx_len, wx_u, wh_u, b_u, wx_c, wh_c, b_c):
    def tc_kernel(x_ref, len_ref, cl_ref, wxu_ref, whu_ref, bu_ref, wxc_ref,
                  whc_ref, bc_ref, out_ref, gx_ref, g2_ref):
        whu = whu_ref[...]
        lenv = len_ref[...]

        x2d = x_ref[...].reshape(W * NROW, D)
        gx_ref[...] = (jnp.dot(x2d, wxu_ref[...],
                               preferred_element_type=jnp.float32) + bu_ref[...])

        h = jnp.zeros((NROW, H), jnp.float32)
        for t in range(W):
            gx = gx_ref[t * NROW:(t + 1) * NROW, :]
            gh = jnp.dot(h, whu, preferred_element_type=jnp.float32)
            r = jax.nn.sigmoid(gx[:, :H] + gh[:, :H])
            z = jax.nn.sigmoid(gx[:, H:2 * H] + gh[:, H:2 * H])
            nn = jnp.tanh(gx[:, 2 * H:] + r * gh[:, 2 * H:])
            h = jnp.where(lenv > t, nn + z * (h - nn), h)

        g2_ref[...] = (jnp.dot(h, wxc_ref[...],
                               preferred_element_type=jnp.float32) + bc_ref[...])
        whc = whc_ref[...]
        clv = cl_ref[...]

        h2 = jnp.zeros((B, H), jnp.float32)
        for s in range(U + 1):
            gx2 = g2_ref[s * B:(s + 1) * B, :]
            gh2 = jnp.dot(h2, whc, preferred_element_type=jnp.float32)
            r2 = jax.nn.sigmoid(gx2[:, :H] + gh2[:, :H])
            z2 = jax.nn.sigmoid(gx2[:, H:2 * H] + gh2[:, H:2 * H])
            n2 = jnp.tanh(gx2[:, 2 * H:] + r2 * gh2[:, 2 * H:])
            h2 = jnp.where(clv >= s, n2 + z2 * (h2 - n2), h2)
        out_ref[...] = h2

    return pl.pallas_call(
        tc_kernel,
        out_shape=jax.ShapeDtypeStruct((B, H), jnp.float32),
        scratch_shapes=[pltpu.VMEM((W * NROW, 3 * H), jnp.float32),
                        pltpu.VMEM((NROW, 3 * H), jnp.float32)],
    )(x_tm, lens, ctx_len, wx_u, wh_u, b_u, wx_c, wh_c, b_c)


def kernel(contexts, context_utterance_lengths, context_lengths, queries,
           query_lengths, emb, Wx_u, Wh_u, b_u, Wx_c, Wh_c, b_c):
    toks = jnp.concatenate([contexts.reshape(-1), queries.reshape(-1)])
    x = _sc_gather(toks, context_lengths, emb)
    x_tm = x.reshape(W, NROW, D)
    lens = jnp.concatenate(
        [context_utterance_lengths, query_lengths[:, None]], axis=1
    ).T.reshape(NROW, 1)
    return _tc_gru(x_tm, lens, context_lengths.reshape(B, 1),
                   Wx_u, Wh_u, b_u.reshape(1, 3 * H),
                   Wx_c, Wh_c, b_c.reshape(1, 3 * H))
